# baseline (speedup 1.0000x reference)
import functools

import jax
import jax.numpy as jnp
from jax import lax
from jax.experimental import pallas as pl
from jax.experimental.pallas import tpu as pltpu
from jax.experimental.pallas import tpu_sc as plsc

NC, NS, LANES = 2, 16, 16
NW = NC * NS
IDXB = 512
NIDX = 1
MAC = IDXB * NIDX
NSLOT = 2


def _body(n_tok, tags, wdim, tdim,
          sent_hbm, wtab_hbm, ttab_hbm, out_hbm,
          idx_v, ttab_v, w_v, t_v, idx_sem, gw_sem, out_sem):
    tok_per_w = n_tok // NW
    nmac = tok_per_w // MAC
    wid = lax.axis_index("s") * NC + lax.axis_index("c")
    base_w = wid * tok_per_w

    pltpu.sync_copy(ttab_hbm, ttab_v)
    lane = lax.broadcasted_iota(jnp.int32, (LANES,), 0)

    def idx_src(g):
        return sent_hbm.at[pl.ds((base_w + g * MAC) // IDXB, NIDX)]

    def out_w_dst(g):
        return out_hbm.at[pl.ds(base_w + g * MAC, MAC), pl.ds(0, wdim)]

    def out_t_dst(g):
        return out_hbm.at[pl.ds(base_w + g * MAC, MAC), pl.ds(wdim, tdim)]

    pltpu.async_copy(idx_src(0), idx_v[0], idx_sem[0])

    def macro(gg, carry):
        for s in range(NSLOT):
            g = gg * NSLOT + s
            pltpu.make_async_copy(idx_src(g), idx_v[s], idx_sem[s]).wait()
            @pl.when(gg > 0)
            def _():
                pltpu.make_async_copy(w_v[s], out_w_dst(g), out_sem[s]).wait()
                pltpu.make_async_copy(t_v[s], out_t_dst(g), out_sem[s]).wait()
            pltpu.async_copy(wtab_hbm.at[idx_v[s].at[0]],
                             w_v[s], gw_sem[s])
            if s == 0:
                pltpu.async_copy(idx_src(g + 1), idx_v[1], idx_sem[1])
            else:
                @pl.when(gg < nmac // NSLOT - 1)
                def _():
                    pltpu.async_copy(idx_src(g + 1), idx_v[0], idx_sem[0])

            def tagloop(bb, carry):
                rows_v = lane + bb * LANES
                tid = lax.rem(idx_v[s][0, pl.ds(bb * LANES, LANES)],
                              jnp.int32(tags))
                for c in range(tdim):
                    colv = jnp.full((LANES,), c, jnp.int32)
                    v = plsc.load_gather(ttab_v, [tid, colv])
                    plsc.store_scatter(t_v[s], [rows_v, colv], v)
                return carry

            lax.fori_loop(0, MAC // LANES, tagloop, 0)

            pltpu.make_async_copy(wtab_hbm.at[idx_v[s].at[0]],
                                  w_v[s], gw_sem[s]).wait()
            pltpu.async_copy(w_v[s], out_w_dst(g), out_sem[s])
            pltpu.async_copy(t_v[s], out_t_dst(g), out_sem[s])
        return carry

    lax.fori_loop(0, nmac // NSLOT, macro, 0)

    for s in range(NSLOT):
        g = nmac - NSLOT + s
        pltpu.make_async_copy(w_v[s], out_w_dst(g), out_sem[s]).wait()
        pltpu.make_async_copy(t_v[s], out_t_dst(g), out_sem[s]).wait()


def _relayout_table(word_table):
    wdim, vocab = word_table.T.shape
    rpb = 2 * wdim
    grid = (vocab + rpb - 1) // rpb

    def tbody(x_ref, o_ref):
        y = x_ref[...].T
        z = y.reshape(wdim, 2, wdim)
        o_ref[:, 0:wdim] = z[:, 0, :]
        o_ref[:, wdim:rpb] = z[:, 1, :]

    out = pl.pallas_call(
        tbody,
        grid=(grid,),
        in_specs=[pl.BlockSpec((wdim, rpb), lambda j: (0, j))],
        out_specs=pl.BlockSpec((wdim, rpb), lambda j: (j, 0)),
        out_shape=jax.ShapeDtypeStruct((vocab // 2, rpb), word_table.dtype),
    )(word_table.T)
    return out.reshape(vocab, wdim)


def kernel(sentence, word_table, tag_table):
    b, l = sentence.shape
    n_tok = b * l
    vocab, wdim = word_table.shape
    tags, tdim = tag_table.shape
    odim = wdim + tdim
    sent = sentence.reshape(n_tok // IDXB, IDXB).astype(jnp.int32)
    word_table = _relayout_table(word_table)

    mesh = plsc.VectorSubcoreMesh(
        core_axis_name="c", subcore_axis_name="s",
        num_cores=NC, num_subcores=NS)
    run = pl.kernel(
        functools.partial(_body, n_tok, tags, wdim, tdim),
        out_type=jax.ShapeDtypeStruct((n_tok, odim), jnp.float32),
        mesh=mesh,
        scratch_types=[
            [pltpu.VMEM((NIDX, IDXB), jnp.int32) for _ in range(NSLOT)],
            pltpu.VMEM((tags, tdim), jnp.float32),
            [pltpu.VMEM((MAC, wdim), jnp.float32) for _ in range(NSLOT)],
            [pltpu.VMEM((MAC, tdim), jnp.float32) for _ in range(NSLOT)],
            [pltpu.SemaphoreType.DMA for _ in range(NSLOT)],
            [pltpu.SemaphoreType.DMA for _ in range(NSLOT)],
            [pltpu.SemaphoreType.DMA for _ in range(NSLOT)],
        ],
        compiler_params=pltpu.CompilerParams(use_tc_tiling_on_sc=False,
                                             needs_layout_passes=False),
    )
    out = run(sent, word_table, tag_table)
    return out.reshape(b, l, odim)


# speedup vs baseline: 2.6393x; 2.6393x over previous
import functools

import jax
import jax.numpy as jnp
from jax import lax
from jax.experimental import pallas as pl
from jax.experimental.pallas import tpu as pltpu
from jax.experimental.pallas import tpu_sc as plsc

NC, NS, LANES = 2, 16, 16
NW = NC * NS
IDXB = 512
NIDX = 1
MAC = IDXB * NIDX
NSLOT = 2


def _body(n_tok, tags, wdim, tdim,
          sent_hbm, wtab_hbm, ttab_hbm, out_hbm,
          idx_v, ttab_v, w_v, t_v, idx_sem, gw_sem, out_sem):
    tok_per_w = n_tok // NW
    nmac = tok_per_w // MAC
    wid = lax.axis_index("s") * NC + lax.axis_index("c")
    base_w = wid * tok_per_w

    pltpu.sync_copy(ttab_hbm, ttab_v)
    lane = lax.broadcasted_iota(jnp.int32, (LANES,), 0)

    def idx_src(g):
        return sent_hbm.at[pl.ds((base_w + g * MAC) // IDXB, NIDX)]

    def out_w_dst(g):
        return out_hbm.at[pl.ds(base_w + g * MAC, MAC), pl.ds(0, wdim)]

    def out_t_dst(g):
        return out_hbm.at[pl.ds(base_w + g * MAC, MAC), pl.ds(wdim, tdim)]

    pltpu.async_copy(idx_src(0), idx_v[0], idx_sem[0])

    def macro(gg, carry):
        for s in range(NSLOT):
            g = gg * NSLOT + s
            pltpu.make_async_copy(idx_src(g), idx_v[s], idx_sem[s]).wait()
            @pl.when(gg > 0)
            def _():
                pltpu.make_async_copy(w_v[s], out_w_dst(g), out_sem[s]).wait()
                pltpu.make_async_copy(t_v[s], out_t_dst(g), out_sem[s]).wait()
            pltpu.async_copy(wtab_hbm.at[idx_v[s].at[0]],
                             w_v[s], gw_sem[s])
            if s == 0:
                pltpu.async_copy(idx_src(g + 1), idx_v[1], idx_sem[1])
            else:
                @pl.when(gg < nmac // NSLOT - 1)
                def _():
                    pltpu.async_copy(idx_src(g + 1), idx_v[0], idx_sem[0])

            def tagloop(bb, carry):
                rows_v = lane + bb * LANES
                tid = lax.rem(idx_v[s][0, pl.ds(bb * LANES, LANES)],
                              jnp.int32(tags))
                for c in range(tdim):
                    colv = jnp.full((LANES,), c, jnp.int32)
                    v = plsc.load_gather(ttab_v, [tid, colv])
                    plsc.store_scatter(t_v[s], [rows_v, colv], v)
                return carry

            lax.fori_loop(0, MAC // LANES, tagloop, 0)

            pltpu.make_async_copy(wtab_hbm.at[idx_v[s].at[0]],
                                  w_v[s], gw_sem[s]).wait()
            pltpu.async_copy(w_v[s], out_w_dst(g), out_sem[s])
            pltpu.async_copy(t_v[s], out_t_dst(g), out_sem[s])
        return carry

    lax.fori_loop(0, nmac // NSLOT, macro, 0)

    for s in range(NSLOT):
        g = nmac - NSLOT + s
        pltpu.make_async_copy(w_v[s], out_w_dst(g), out_sem[s]).wait()
        pltpu.make_async_copy(t_v[s], out_t_dst(g), out_sem[s]).wait()


def kernel(sentence, word_table, tag_table):
    b, l = sentence.shape
    n_tok = b * l
    vocab, wdim = word_table.shape
    tags, tdim = tag_table.shape
    odim = wdim + tdim
    sent = sentence.reshape(n_tok // IDXB, IDXB).astype(jnp.int32)

    mesh = plsc.VectorSubcoreMesh(
        core_axis_name="c", subcore_axis_name="s",
        num_cores=NC, num_subcores=NS)
    run = pl.kernel(
        functools.partial(_body, n_tok, tags, wdim, tdim),
        out_type=jax.ShapeDtypeStruct((n_tok, odim), jnp.float32),
        mesh=mesh,
        scratch_types=[
            [pltpu.VMEM((NIDX, IDXB), jnp.int32) for _ in range(NSLOT)],
            pltpu.VMEM((tags, tdim), jnp.float32),
            [pltpu.VMEM((MAC, wdim), jnp.float32) for _ in range(NSLOT)],
            [pltpu.VMEM((MAC, tdim), jnp.float32) for _ in range(NSLOT)],
            [pltpu.SemaphoreType.DMA for _ in range(NSLOT)],
            [pltpu.SemaphoreType.DMA for _ in range(NSLOT)],
            [pltpu.SemaphoreType.DMA for _ in range(NSLOT)],
        ],
        compiler_params=pltpu.CompilerParams(use_tc_tiling_on_sc=False,
                                             needs_layout_passes=False),
    )
    out = run(sent, word_table, tag_table)
    return out.reshape(b, l, odim)
